# baseline (speedup 1.0000x reference)
import functools

import jax
import jax.numpy as jnp
from jax import lax
from jax.experimental import pallas as pl
from jax.experimental.pallas import tpu as pltpu
from jax.experimental.pallas import tpu_sc as plsc

VOCAB = 100000
EMBED = 128
BATCH = 1024

_NC = 2
_NS = 16
_NW = _NC * _NS
_BPW = BATCH // _NW

_VT = 2048
_NFULL = VOCAB // _VT
_NBUF = 4


def _sc_gather(emb_table, idx):
    mesh = plsc.VectorSubcoreMesh(core_axis_name="c", subcore_axis_name="s")

    @functools.partial(
        pl.kernel,
        mesh=mesh,
        out_type=jax.ShapeDtypeStruct((BATCH, EMBED), jnp.float32),
        scratch_types=[
            pltpu.VMEM((_BPW,), jnp.int32),
            pltpu.VMEM((_BPW, EMBED), jnp.float32),
            pltpu.SemaphoreType.DMA,
        ],
    )
    def gather_kernel(table_hbm, idx_hbm, out_hbm, idx_v, rows_v, sem):
        wid = lax.axis_index("s") * _NC + lax.axis_index("c")
        base = wid * _BPW
        pltpu.sync_copy(idx_hbm.at[pl.ds(base, _BPW)], idx_v)
        pltpu.async_copy(table_hbm.at[idx_v], rows_v, sem).wait()
        pltpu.sync_copy(rows_v, out_hbm.at[pl.ds(base, _BPW)])

    return gather_kernel(emb_table, idx)


def _mm_ring_body(x_ref, w_ref, b_ref, out_hbm, slots, sems):
    j = pl.program_id(0)
    jm = lax.rem(j, _NBUF)

    @pl.when(j >= _NBUF)
    def _drain_slot():
        pltpu.make_async_copy(
            slots.at[jm],
            out_hbm.at[:, pl.ds((j - _NBUF) * _VT, _VT)],
            sems.at[jm],
        ).wait()

    x = jnp.maximum(x_ref[...], 0.0)
    slots[jm] = lax.dot_general(
        x, w_ref[...], (((1,), (1,)), ((), ())),
        preferred_element_type=jnp.float32,
    ) + b_ref[...]
    pltpu.make_async_copy(
        slots.at[jm],
        out_hbm.at[:, pl.ds(j * _VT, _VT)],
        sems.at[jm],
    ).start()

    @pl.when(j == _NFULL - 1)
    def _drain_all():
        for b in range(_NBUF):
            jj = j - (_NBUF - 1) + b
            bm = lax.rem(jj, _NBUF)
            pltpu.make_async_copy(
                slots.at[bm],
                out_hbm.at[:, pl.ds(jj * _VT, _VT)],
                sems.at[bm],
            ).wait()


def _tail_body(big_ref, x_ref, w_ref, b_ref, o_ref):
    x = jnp.maximum(x_ref[...], 0.0)
    o_ref[...] = lax.dot_general(
        x, w_ref[...], (((1,), (1,)), ((), ())),
        preferred_element_type=jnp.float32,
    ) + b_ref[...]


def _tc_project(x, fc_w, fc_b2d):
    big = pl.pallas_call(
        _mm_ring_body,
        grid=(_NFULL,),
        in_specs=[
            pl.BlockSpec((BATCH, EMBED), lambda j: (0, 0)),
            pl.BlockSpec((_VT, EMBED), lambda j: (j, 0)),
            pl.BlockSpec((1, _VT), lambda j: (0, j)),
        ],
        out_specs=pl.BlockSpec(memory_space=pl.ANY),
        out_shape=jax.ShapeDtypeStruct((BATCH, VOCAB), jnp.float32),
        scratch_shapes=[
            pltpu.VMEM((_NBUF, BATCH, _VT), jnp.float32),
            pltpu.SemaphoreType.DMA((_NBUF,)),
        ],
    )(x, fc_w, fc_b2d)

    return pl.pallas_call(
        _tail_body,
        grid=(1,),
        in_specs=[
            pl.BlockSpec(memory_space=pl.ANY),
            pl.BlockSpec((BATCH, EMBED), lambda i: (0, 0)),
            pl.BlockSpec((_VT, EMBED), lambda i: (_NFULL, 0)),
            pl.BlockSpec((1, _VT), lambda i: (0, _NFULL)),
        ],
        out_specs=pl.BlockSpec((BATCH, _VT), lambda i: (0, _NFULL)),
        out_shape=jax.ShapeDtypeStruct((BATCH, VOCAB), jnp.float32),
        input_output_aliases={0: 0},
    )(big, x, fc_w, fc_b2d)


def kernel(text, emb_table, fc_w, fc_b):
    idx = text.astype(jnp.int32)
    x = _sc_gather(emb_table, idx)
    return _tc_project(x, fc_w, fc_b.reshape(1, VOCAB))


# speedup vs baseline: 1.0012x; 1.0012x over previous
import functools

import jax
import jax.numpy as jnp
from jax import lax
from jax.experimental import pallas as pl
from jax.experimental.pallas import tpu as pltpu
from jax.experimental.pallas import tpu_sc as plsc

VOCAB = 100000
EMBED = 128
BATCH = 1024

_NC = 2
_NS = 16
_NW = _NC * _NS
_BPW = BATCH // _NW

_VT = 2048
_NFULL = VOCAB // _VT
_NBUF = 4


def _sc_gather(emb_table, idx):
    mesh = plsc.VectorSubcoreMesh(core_axis_name="c", subcore_axis_name="s")

    @functools.partial(
        pl.kernel,
        mesh=mesh,
        out_type=jax.ShapeDtypeStruct((BATCH, EMBED), jnp.float32),
        scratch_types=[
            pltpu.VMEM((_BPW,), jnp.int32),
            pltpu.VMEM((_BPW, EMBED), jnp.float32),
            pltpu.SemaphoreType.DMA,
        ],
    )
    def gather_kernel(table_hbm, idx_hbm, out_hbm, idx_v, rows_v, sem):
        wid = lax.axis_index("s") * _NC + lax.axis_index("c")
        base = wid * _BPW
        pltpu.sync_copy(idx_hbm.at[pl.ds(base, _BPW)], idx_v)
        pltpu.async_copy(table_hbm.at[idx_v], rows_v, sem).wait()
        pltpu.sync_copy(rows_v, out_hbm.at[pl.ds(base, _BPW)])

    return gather_kernel(emb_table, idx)


def _mm_ring_body(x_ref, w_ref, b_ref, out_hbm, *slots_and_sems):
    slots = slots_and_sems[:_NBUF]
    sems = slots_and_sems[_NBUF:]
    j = pl.program_id(0)
    jm = lax.rem(j, _NBUF)

    x = jnp.maximum(x_ref[...], 0.0)
    o = lax.dot_general(
        x, w_ref[...], (((1,), (1,)), ((), ())),
        preferred_element_type=jnp.float32,
    ) + b_ref[...]

    for b in range(_NBUF):
        @pl.when(jnp.logical_and(jm == b, j >= _NBUF))
        def _drain_slot(b=b):
            pltpu.make_async_copy(
                slots[b],
                out_hbm.at[:, pl.ds((j - _NBUF) * _VT, _VT)],
                sems[b],
            ).wait()

        @pl.when(jm == b)
        def _emit(b=b):
            slots[b][...] = o
            pltpu.make_async_copy(
                slots[b],
                out_hbm.at[:, pl.ds(j * _VT, _VT)],
                sems[b],
            ).start()

    @pl.when(j == _NFULL - 1)
    def _drain_all():
        for b in range(_NBUF):
            jj = _NFULL - _NBUF + b
            pltpu.make_async_copy(
                slots[jj % _NBUF],
                out_hbm.at[:, pl.ds(jj * _VT, _VT)],
                sems[jj % _NBUF],
            ).wait()


def _tail_body(big_ref, x_ref, w_ref, b_ref, o_ref):
    x = jnp.maximum(x_ref[...], 0.0)
    o_ref[...] = lax.dot_general(
        x, w_ref[...], (((1,), (1,)), ((), ())),
        preferred_element_type=jnp.float32,
    ) + b_ref[...]


def _tc_project(x, fc_w, fc_b2d):
    big = pl.pallas_call(
        _mm_ring_body,
        grid=(_NFULL,),
        in_specs=[
            pl.BlockSpec((BATCH, EMBED), lambda j: (0, 0)),
            pl.BlockSpec((_VT, EMBED), lambda j: (j, 0)),
            pl.BlockSpec((1, _VT), lambda j: (0, j)),
        ],
        out_specs=pl.BlockSpec(memory_space=pl.ANY),
        out_shape=jax.ShapeDtypeStruct((BATCH, VOCAB), jnp.float32),
        scratch_shapes=(
            [pltpu.VMEM((BATCH, _VT), jnp.float32) for _ in range(_NBUF)]
            + [pltpu.SemaphoreType.DMA for _ in range(_NBUF)]
        ),
    )(x, fc_w, fc_b2d)

    return pl.pallas_call(
        _tail_body,
        grid=(1,),
        in_specs=[
            pl.BlockSpec(memory_space=pl.ANY),
            pl.BlockSpec((BATCH, EMBED), lambda i: (0, 0)),
            pl.BlockSpec((_VT, EMBED), lambda i: (_NFULL, 0)),
            pl.BlockSpec((1, _VT), lambda i: (0, _NFULL)),
        ],
        out_specs=pl.BlockSpec((BATCH, _VT), lambda i: (0, _NFULL)),
        out_shape=jax.ShapeDtypeStruct((BATCH, VOCAB), jnp.float32),
        input_output_aliases={0: 0},
    )(big, x, fc_w, fc_b2d)


def kernel(text, emb_table, fc_w, fc_b):
    idx = text.astype(jnp.int32)
    x = _sc_gather(emb_table, idx)
    return _tc_project(x, fc_w, fc_b.reshape(1, VOCAB))


# speedup vs baseline: 2.3823x; 2.3793x over previous
import functools

import jax
import jax.numpy as jnp
from jax import lax
from jax.experimental import pallas as pl
from jax.experimental.pallas import tpu as pltpu
from jax.experimental.pallas import tpu_sc as plsc

VOCAB = 100000
EMBED = 128
BATCH = 1024

_NC = 2
_NS = 16
_NW = _NC * _NS
_BPW = BATCH // _NW

_VT = 2048
_NSTEPS = pl.cdiv(VOCAB, _VT)


def _sc_gather(emb_table, idx):
    mesh = plsc.VectorSubcoreMesh(core_axis_name="c", subcore_axis_name="s")

    @functools.partial(
        pl.kernel,
        mesh=mesh,
        out_type=jax.ShapeDtypeStruct((BATCH, EMBED), jnp.float32),
        scratch_types=[
            pltpu.VMEM((_BPW,), jnp.int32),
            pltpu.VMEM((_BPW, EMBED), jnp.float32),
            pltpu.SemaphoreType.DMA,
        ],
    )
    def gather_kernel(table_hbm, idx_hbm, out_hbm, idx_v, rows_v, sem):
        wid = lax.axis_index("s") * _NC + lax.axis_index("c")
        base = wid * _BPW
        pltpu.sync_copy(idx_hbm.at[pl.ds(base, _BPW)], idx_v)
        pltpu.async_copy(table_hbm.at[idx_v], rows_v, sem).wait()
        pltpu.sync_copy(rows_v, out_hbm.at[pl.ds(base, _BPW)])

    return gather_kernel(emb_table, idx)


def _mmT_body(x_ref, w_ref, b_ref, o_ref):
    x = jnp.maximum(x_ref[...], 0.0)
    o_ref[...] = lax.dot_general(
        w_ref[...], x, (((1,), (1,)), ((), ())),
        preferred_element_type=jnp.float32,
    ) + b_ref[...]


def _tc_project_t(x, fc_w, fc_b2d):
    return pl.pallas_call(
        _mmT_body,
        grid=(_NSTEPS,),
        in_specs=[
            pl.BlockSpec((BATCH, EMBED), lambda j: (0, 0)),
            pl.BlockSpec((_VT, EMBED), lambda j: (j, 0)),
            pl.BlockSpec((_VT, 1), lambda j: (j, 0)),
        ],
        out_specs=pl.BlockSpec((_VT, BATCH), lambda j: (j, 0)),
        out_shape=jax.ShapeDtypeStruct((VOCAB, BATCH), jnp.float32),
    )(x, fc_w, fc_b2d)


def kernel(text, emb_table, fc_w, fc_b):
    idx = text.astype(jnp.int32)
    x = _sc_gather(emb_table, idx)
    out_t = _tc_project_t(x, fc_w, fc_b.reshape(VOCAB, 1))
    return out_t.T
